# baseline (speedup 1.0000x reference)
import functools

import jax
import jax.numpy as jnp
from jax.experimental import pallas as pl
from jax.experimental.pallas import tpu as pltpu

_NEG_SLOPE = 0.01
_BN_EPS = 1e-5
_D_TILE = 8


def _conv_stats_kernel(xhw_ref, wk_ref, y_ref, stats_ref, *, d_tile, hw):
    mt = pl.program_id(1)
    ts = d_tile * hw
    acc = None
    for a in range(3):
        xs = xhw_ref[:, pl.ds((mt * d_tile + a) * hw, ts)]
        p = jax.lax.dot_general(
            xs, wk_ref[a],
            dimension_numbers=(((0,), (0,)), ((), ())),
            preferred_element_type=jnp.float32)
        acc = p if acc is None else acc + p
    yt = acc.T
    y_ref[...] = yt.astype(y_ref.dtype)
    stats_ref[:, 0:1] = jnp.sum(yt, axis=-1, keepdims=True)
    stats_ref[:, 1:2] = jnp.sum(yt * yt, axis=-1, keepdims=True)


def _bn_act_kernel(y_ref, scale_ref, shift_ref, o_ref):
    y = y_ref[...].astype(jnp.float32) * scale_ref[...] + shift_ref[...]
    o_ref[...] = jnp.where(y >= 0.0, y, _NEG_SLOPE * y)


def kernel(w, gamma, beta, seqs, seqL):
    del seqL
    N, C, D, H, W = seqs.shape
    Cout = w.shape[0]
    hw = H * W
    ms = D * hw
    n_tiles = D // _D_TILE
    ts = _D_TILE * hw

    xb = seqs.astype(jnp.bfloat16)
    xq = jnp.pad(xb, ((0, 0), (0, 0), (1, 1), (1, 1), (1, 1)))
    taps = [xq[:, :, :, b:b + H, c:c + W].reshape(N, C, (D + 2) * hw)
            for b in range(3) for c in range(3)]
    xhw = jnp.concatenate(taps, axis=1)

    wk = w.transpose(2, 3, 4, 1, 0).reshape(3, 9 * C, Cout).astype(jnp.bfloat16)

    cost1 = pl.CostEstimate(
        flops=2 * N * ms * (27 * C) * Cout,
        transcendentals=0,
        bytes_accessed=(N * 9 * C * (D + 2) * hw * 2 + 3 * 9 * C * Cout * 2
                       + N * Cout * ms * 2 + N * n_tiles * Cout * 2 * 4))

    y_cm, part_stats = pl.pallas_call(
        functools.partial(_conv_stats_kernel, d_tile=_D_TILE, hw=hw),
        out_shape=(jax.ShapeDtypeStruct((N, Cout, ms), jnp.bfloat16),
                   jax.ShapeDtypeStruct((N, n_tiles, Cout, 2), jnp.float32)),
        grid=(N, n_tiles),
        in_specs=[pl.BlockSpec((None, 9 * C, (D + 2) * hw), lambda n, i: (n, 0, 0)),
                  pl.BlockSpec((3, 9 * C, Cout), lambda n, i: (0, 0, 0))],
        out_specs=[pl.BlockSpec((None, Cout, ts), lambda n, i: (n, 0, i)),
                   pl.BlockSpec((None, None, Cout, 2), lambda n, i: (n, i, 0, 0))],
        compiler_params=pltpu.CompilerParams(
            dimension_semantics=("parallel", "arbitrary")),
        cost_estimate=cost1,
    )(xhw, wk)

    M = N * ms
    stats = jnp.sum(part_stats, axis=(0, 1))
    mean = stats[:, 0] / M
    var = jnp.maximum(stats[:, 1] / M - mean * mean, 0.0)
    scale = gamma / jnp.sqrt(var + _BN_EPS)
    shift = beta - mean * scale
    scale_c = scale.reshape(Cout, 1).astype(jnp.float32)
    shift_c = shift.reshape(Cout, 1).astype(jnp.float32)

    cost2 = pl.CostEstimate(
        flops=4 * N * ms * Cout,
        transcendentals=0,
        bytes_accessed=N * ms * Cout * (2 + 4) + 2 * Cout * 4)

    out_cm = pl.pallas_call(
        _bn_act_kernel,
        out_shape=jax.ShapeDtypeStruct((N, Cout, ms), jnp.float32),
        grid=(N, n_tiles),
        in_specs=[pl.BlockSpec((None, Cout, ts), lambda n, i: (n, 0, i)),
                  pl.BlockSpec((Cout, 1), lambda n, i: (0, 0)),
                  pl.BlockSpec((Cout, 1), lambda n, i: (0, 0))],
        out_specs=pl.BlockSpec((None, Cout, ts), lambda n, i: (n, 0, i)),
        compiler_params=pltpu.CompilerParams(
            dimension_semantics=("parallel", "parallel")),
        cost_estimate=cost2,
    )(y_cm, scale_c, shift_c)

    return out_cm.reshape(N, Cout, D, H, W)


# speedup vs baseline: 2.8563x; 2.8563x over previous
import functools

import jax
import jax.numpy as jnp
from jax.experimental import pallas as pl
from jax.experimental.pallas import tpu as pltpu

_NEG_SLOPE = 0.01
_BN_EPS = 1e-5
_D_TILE = 8
_PLANE = 1152
_LEAD = 128


def _conv_stats_kernel(xb_ref, wk_ref, y_ref, stats_ref, col_ref, *,
                       c_in, d_tile, hw):
    mt = pl.program_id(1)
    ts2 = d_tile * _PLANE
    span = ts2 + 2 * _PLANE
    base = _LEAD + mt * ts2

    w_pos = jax.lax.broadcasted_iota(jnp.int32, (c_in, span), 1) % 32
    vwin = xb_ref[:, pl.ds(mt * ts2, span + 2 * _LEAD)]
    for b in range(3):
        for cw in range(3):
            t = b * 3 + cw
            off = _LEAD + b * 32 + (cw - 1)
            v = vwin[:, off:off + span]
            if cw == 0:
                v = jnp.where(w_pos == 0, 0.0, v)
            elif cw == 2:
                v = jnp.where(w_pos == 31, 0.0, v)
            col_ref[t * c_in:(t + 1) * c_in, :] = v.astype(jnp.bfloat16)

    acc = None
    for a in range(3):
        xs = col_ref[:, a * _PLANE:a * _PLANE + ts2]
        p = jax.lax.dot_general(
            xs, wk_ref[a],
            dimension_numbers=(((0,), (0,)), ((), ())),
            preferred_element_type=jnp.float32)
        acc = p if acc is None else acc + p

    yt = acc.T
    yd = jnp.concatenate(
        [yt[:, dd * _PLANE:dd * _PLANE + hw] for dd in range(d_tile)], axis=1)
    y_ref[...] = yd.astype(y_ref.dtype)
    stats_ref[:, 0:1] = jnp.sum(yd, axis=-1, keepdims=True)
    stats_ref[:, 1:2] = jnp.sum(yd * yd, axis=-1, keepdims=True)


def _bn_act_kernel(y_ref, scale_ref, shift_ref, o_ref):
    y = y_ref[...].astype(jnp.float32) * scale_ref[...] + shift_ref[...]
    o_ref[...] = jnp.where(y >= 0.0, y, _NEG_SLOPE * y)


def kernel(w, gamma, beta, seqs, seqL):
    del seqL
    N, C, D, H, W = seqs.shape
    Cout = w.shape[0]
    hw = H * W
    ms = D * hw
    n_tiles = D // _D_TILE
    ts = _D_TILE * hw

    L = _LEAD + (D + 2) * _PLANE + 128
    xpad = jnp.pad(seqs, ((0, 0), (0, 0), (1, 1), (1, 1), (0, 0)))
    xpad = xpad.reshape(N, C, D + 2, (H + 2) * W)
    xpad = jnp.pad(xpad, ((0, 0), (0, 0), (0, 0), (0, _PLANE - (H + 2) * W)))
    xb = xpad.reshape(N, C, (D + 2) * _PLANE)
    xb = jnp.pad(xb, ((0, 0), (0, 0), (_LEAD, L - _LEAD - (D + 2) * _PLANE)))

    wk = w.transpose(2, 3, 4, 1, 0).reshape(3, 9 * C, Cout).astype(jnp.bfloat16)

    cost1 = pl.CostEstimate(
        flops=2 * N * ms * (27 * C) * Cout,
        transcendentals=0,
        bytes_accessed=(N * C * L * 4 + 3 * 9 * C * Cout * 2
                       + N * Cout * ms * 2 + N * n_tiles * Cout * 2 * 4))

    y_cm, part_stats = pl.pallas_call(
        functools.partial(_conv_stats_kernel, c_in=C, d_tile=_D_TILE, hw=hw),
        out_shape=(jax.ShapeDtypeStruct((N, Cout, ms), jnp.bfloat16),
                   jax.ShapeDtypeStruct((N, n_tiles, Cout, 2), jnp.float32)),
        grid=(N, n_tiles),
        in_specs=[pl.BlockSpec((None, C, L), lambda n, i: (n, 0, 0)),
                  pl.BlockSpec((3, 9 * C, Cout), lambda n, i: (0, 0, 0))],
        out_specs=[pl.BlockSpec((None, Cout, ts), lambda n, i: (n, 0, i)),
                   pl.BlockSpec((None, None, Cout, 2), lambda n, i: (n, i, 0, 0))],
        scratch_shapes=[pltpu.VMEM((9 * C, _D_TILE * _PLANE + 2 * _PLANE),
                                   jnp.bfloat16)],
        compiler_params=pltpu.CompilerParams(
            dimension_semantics=("parallel", "arbitrary")),
        cost_estimate=cost1,
    )(xb, wk)

    M = N * ms
    stats = jnp.sum(part_stats, axis=(0, 1))
    mean = stats[:, 0] / M
    var = jnp.maximum(stats[:, 1] / M - mean * mean, 0.0)
    scale = gamma / jnp.sqrt(var + _BN_EPS)
    shift = beta - mean * scale
    scale_c = scale.reshape(Cout, 1).astype(jnp.float32)
    shift_c = shift.reshape(Cout, 1).astype(jnp.float32)

    cost2 = pl.CostEstimate(
        flops=4 * N * ms * Cout,
        transcendentals=0,
        bytes_accessed=N * ms * Cout * (2 + 4) + 2 * Cout * 4)

    out_cm = pl.pallas_call(
        _bn_act_kernel,
        out_shape=jax.ShapeDtypeStruct((N, Cout, ms), jnp.float32),
        grid=(N, n_tiles),
        in_specs=[pl.BlockSpec((None, Cout, ts), lambda n, i: (n, 0, i)),
                  pl.BlockSpec((Cout, 1), lambda n, i: (0, 0)),
                  pl.BlockSpec((Cout, 1), lambda n, i: (0, 0))],
        out_specs=pl.BlockSpec((None, Cout, ts), lambda n, i: (n, 0, i)),
        compiler_params=pltpu.CompilerParams(
            dimension_semantics=("parallel", "parallel")),
        cost_estimate=cost2,
    )(y_cm, scale_c, shift_c)

    return out_cm.reshape(N, Cout, D, H, W)
